# baseline (speedup 1.0000x reference)
import jax
import jax.numpy as jnp
from jax.experimental import pallas as pl
from jax.experimental.pallas import tpu as pltpu


def _body(row_ref, col_ref, out_ref, *, height, width, num_pos_feats, nb):
    col = col_ref[:width, :]
    rows = row_ref[:height, :]
    out_ref[:, :, :, :num_pos_feats] = jnp.broadcast_to(
        col[None, None, :, :], (nb, height, width, num_pos_feats))
    out_ref[:, :, :, num_pos_feats:] = jnp.broadcast_to(
        rows[None, :, None, :], (nb, height, width, num_pos_feats))


def kernel(pixel_values, pixel_mask, row_embed, col_embed):
    batch = pixel_values.shape[0]
    height, width = pixel_values.shape[-2:]
    num_rows, num_pos_feats = row_embed.shape
    channels = 2 * num_pos_feats
    nb = 4
    grid = (batch // nb,)

    import functools
    body = functools.partial(
        _body, height=height, width=width, num_pos_feats=num_pos_feats,
        nb=nb)

    out = pl.pallas_call(
        body,
        grid=grid,
        in_specs=[
            pl.BlockSpec((num_rows, num_pos_feats), lambda i: (0, 0)),
            pl.BlockSpec((num_rows, num_pos_feats), lambda i: (0, 0)),
        ],
        out_specs=pl.BlockSpec((nb, height, width, channels),
                               lambda i: (i, 0, 0, 0)),
        out_shape=jax.ShapeDtypeStruct((batch, height, width, channels),
                                       jnp.float32),
        compiler_params=pltpu.CompilerParams(
            dimension_semantics=("parallel",)),
    )(row_embed, col_embed)
    return jnp.transpose(out, (0, 3, 1, 2))


# speedup vs baseline: 1.1215x; 1.1215x over previous
import jax
import jax.numpy as jnp
from jax.experimental import pallas as pl
from jax.experimental.pallas import tpu as pltpu


def _body(row_ref, col_ref, out_ref, *, height, width, num_pos_feats, nb):
    col = col_ref[:width, :]
    rows = row_ref[:height, :]
    out_ref[:, :, :, :num_pos_feats] = jnp.broadcast_to(
        col[None, None, :, :], (nb, height, width, num_pos_feats))
    out_ref[:, :, :, num_pos_feats:] = jnp.broadcast_to(
        rows[None, :, None, :], (nb, height, width, num_pos_feats))


def kernel(pixel_values, pixel_mask, row_embed, col_embed):
    batch = pixel_values.shape[0]
    height, width = pixel_values.shape[-2:]
    num_rows, num_pos_feats = row_embed.shape
    channels = 2 * num_pos_feats
    nb = 2
    grid = (batch // nb,)

    import functools
    body = functools.partial(
        _body, height=height, width=width, num_pos_feats=num_pos_feats,
        nb=nb)

    out = pl.pallas_call(
        body,
        grid=grid,
        in_specs=[
            pl.BlockSpec((num_rows, num_pos_feats), lambda i: (0, 0)),
            pl.BlockSpec((num_rows, num_pos_feats), lambda i: (0, 0)),
        ],
        out_specs=pl.BlockSpec((nb, height, width, channels),
                               lambda i: (i, 0, 0, 0)),
        out_shape=jax.ShapeDtypeStruct((batch, height, width, channels),
                                       jnp.float32),
        compiler_params=pltpu.CompilerParams(
            dimension_semantics=("parallel",)),
    )(row_embed, col_embed)
    return jnp.transpose(out, (0, 3, 1, 2))


# speedup vs baseline: 1.1232x; 1.0015x over previous
import functools

import jax
import jax.numpy as jnp
from jax.experimental import pallas as pl
from jax.experimental.pallas import tpu as pltpu


def _body(row_ref, col_ref, out_ref, *, height, width, num_pos_feats, nb):
    col = col_ref[:width, :]
    rows = row_ref[:height, :]
    out_ref[:, :, :, :num_pos_feats] = jnp.broadcast_to(
        col[None, None, :, :], (nb, height, width, num_pos_feats))
    out_ref[:, :, :, num_pos_feats:] = jnp.broadcast_to(
        rows[None, :, None, :], (nb, height, width, num_pos_feats))


def kernel(pixel_values, pixel_mask, row_embed, col_embed):
    batch = pixel_values.shape[0]
    height, width = pixel_values.shape[-2:]
    num_rows, num_pos_feats = row_embed.shape
    channels = 2 * num_pos_feats
    nb = 2
    grid = (batch // nb,)

    body = functools.partial(
        _body, height=height, width=width, num_pos_feats=num_pos_feats,
        nb=nb)

    out = pl.pallas_call(
        body,
        grid=grid,
        in_specs=[
            pl.BlockSpec((num_rows, num_pos_feats), lambda i: (0, 0)),
            pl.BlockSpec((num_rows, num_pos_feats), lambda i: (0, 0)),
        ],
        out_specs=pl.BlockSpec((nb, height, width, channels),
                               lambda i: (i, 0, 0, 0)),
        out_shape=jax.ShapeDtypeStruct((batch, height, width, channels),
                                       jnp.float32),
        compiler_params=pltpu.CompilerParams(
            dimension_semantics=("parallel",)),
    )(row_embed, col_embed)
    return jnp.transpose(out, (0, 3, 1, 2))
